# baseline (speedup 1.0000x reference)
import jax
import jax.numpy as jnp
from jax.experimental import pallas as pl
from jax.experimental.pallas import tpu as pltpu


def kernel(inputs):
    B, T, D = inputs.shape

    flat = inputs.reshape(B, T * D)

    def body(in_ref, out_ref):
        out_ref[...] = in_ref[...]

    return pl.pallas_call(
        body,
        out_shape=jax.ShapeDtypeStruct((B, D), inputs.dtype),
        grid=(1,),
        in_specs=[pl.BlockSpec((B, D), lambda i: (0, 0))],
        out_specs=pl.BlockSpec((B, D), lambda i: (0, 0)),
    )(flat)


# speedup vs baseline: 95.2445x; 95.2445x over previous
import jax
import jax.numpy as jnp
from jax.experimental import pallas as pl
from jax.experimental.pallas import tpu as pltpu


def kernel(inputs):
    B, T, D = inputs.shape

    def body(in_ref, out_ref, sem):
        pltpu.make_async_copy(in_ref.at[:, 0, :], out_ref, sem).start()
        pltpu.make_async_copy(in_ref.at[:, 0, :], out_ref, sem).wait()

    return pl.pallas_call(
        body,
        out_shape=jax.ShapeDtypeStruct((B, D), inputs.dtype),
        in_specs=[pl.BlockSpec(memory_space=pl.ANY)],
        out_specs=pl.BlockSpec(memory_space=pl.ANY),
        scratch_shapes=[pltpu.SemaphoreType.DMA],
        compiler_params=pltpu.CompilerParams(
            skip_device_barrier=True,
            disable_bounds_checks=True,
            disable_semaphore_checks=True,
        ),
    )(inputs)
